# baseline (device time: 17759 ns/iter reference)
import os

import numpy as np

import jax
import jax.numpy as jnp
from jax import lax
from jax.experimental import pallas as pl
from jax.experimental.pallas import tpu as pltpu

N_DEV = 4
B, SQ, D = 2, 128, 512
DH = 64
CH = (B * SQ) // N_DEV
CPB = SQ // CH


def kernel(x, Wq, Wk, Wv, Wo):
    d_local = Wq.shape[1]
    n_heads = d_local // DH
    n_rows = B * SQ

    Wq_b = Wq.astype(jnp.bfloat16)
    Wk_b = Wk.astype(jnp.bfloat16)
    Wv_b = Wv.astype(jnp.bfloat16)
    Wo_b = Wo.astype(jnp.bfloat16)

    def body(x_ref, wq_ref, wk_ref, wv_ref, wo_ref,
             out_ref, ctx_ref, wbd_ref, pbuf_ref, rs_ref, ag_ref,
             rs_send, rs_recv, ag_send, ag_recv):
        my = lax.axis_index("i")

        barrier_sem = pltpu.get_barrier_semaphore()
        for k in (1, N_DEV - 1):
            pl.semaphore_signal(
                barrier_sem, inc=1,
                device_id=((my + k) % N_DEV,),
                device_id_type=pl.DeviceIdType.MESH,
            )
        pl.semaphore_wait(barrier_sem, 2)

        f32 = jnp.float32
        bf16 = jnp.bfloat16

        pos = lax.broadcasted_iota(jnp.int32, (SQ, d_local), 0).astype(f32)
        d_i = lax.broadcasted_iota(jnp.int32, (SQ, d_local), 1) % DH
        pair = (d_i // 2).astype(f32)
        inv = jnp.exp(pair * (-2.0 * np.log(10000.0) / DH))
        ang = pos * inv
        cos_t = jnp.cos(ang).astype(bf16)
        sgn = jnp.where(d_i % 2 == 0, -1.0, 1.0).astype(f32)
        sin_t = (jnp.sin(ang) * sgn).astype(bf16)
        pi = lax.broadcasted_iota(jnp.int32, (d_local, d_local), 0)
        pj = lax.broadcasted_iota(jnp.int32, (d_local, d_local), 1)
        P = (pj == pi + 1 - 2 * (pi % 2)).astype(bf16)

        if os.environ.get("MINIMAL") == "1":
            out_ref[0] = x_ref[0].astype(bf16)
            out_ref[1] = x_ref[1].astype(bf16)
            return

        def rope(t):
            sw = jax.lax.dot(t, P, preferred_element_type=f32).astype(bf16)
            return t * cos_t + sw * sin_t

        wqb = wq_ref[:, :].astype(bf16)
        wkb = wk_ref[:, :].astype(bf16)
        wvb = wv_ref[:, :].astype(bf16)
        wob = wo_ref[:, :].astype(bf16)

        skip_comm = os.environ.get("SKIP_COMM") == "1"

        dn = (((1,), (1,)), ((), ()))
        for b in range(B):
            xb = x_ref[b].astype(bf16)
            q = jax.lax.dot(xb, wqb, preferred_element_type=f32).astype(bf16)
            k = jax.lax.dot(xb, wkb, preferred_element_type=f32).astype(bf16)
            vb = jax.lax.dot(xb, wvb, preferred_element_type=f32).astype(bf16)
            qr = rope(q) * jnp.asarray(0.125, bf16)
            kr = rope(k)

            heads = [slice(h * DH, (h + 1) * DH) for h in range(n_heads)]
            q_st = jnp.concatenate([qr[:, c] for c in heads], axis=0)
            k_st = jnp.concatenate([kr[:, c] for c in heads], axis=0)
            s_all = lax.dot_general(q_st, k_st, dn,
                                    preferred_element_type=f32)
            s_diag = jnp.concatenate(
                [s_all[h * SQ:(h + 1) * SQ, h * SQ:(h + 1) * SQ]
                 for h in range(n_heads)], axis=0)
            e = jnp.exp(s_diag)
            w = (e / jnp.sum(e, axis=-1, keepdims=True)).astype(bf16)
            wbd_ref[:, :] = jnp.zeros((n_heads * SQ, n_heads * SQ), bf16)
            for h in range(n_heads):
                wbd_ref[h * SQ:(h + 1) * SQ, h * SQ:(h + 1) * SQ] = (
                    w[h * SQ:(h + 1) * SQ, :])
            v_st = jnp.concatenate([vb[:, c] for c in heads], axis=0)
            ctx_st = jax.lax.dot(wbd_ref[:, :], v_st,
                                 preferred_element_type=f32).astype(bf16)
            for h in range(n_heads):
                ctx_ref[b * SQ:(b + 1) * SQ, heads[h]] = (
                    ctx_st[h * SQ:(h + 1) * SQ, :])

            partial_b = jax.lax.dot(ctx_ref[b * SQ:(b + 1) * SQ, :], wob,
                                    preferred_element_type=f32)
            for j in range(CPB):
                c = b * CPB + j
                pbuf_ref[c] = partial_b[j * CH:(j + 1) * CH, :].astype(bf16)
                if skip_comm:
                    out_ref[b, j * CH:(j + 1) * CH, :] = pbuf_ref[c][:, :]
                else:
                    @pl.when(my != c)
                    def _():
                        pltpu.make_async_remote_copy(
                            src_ref=pbuf_ref.at[c],
                            dst_ref=rs_ref.at[my],
                            send_sem=rs_send.at[c],
                            recv_sem=rs_recv.at[my],
                            device_id=(c,),
                            device_id_type=pl.DeviceIdType.MESH,
                        ).start()

        if skip_comm:
            return

        for k_ in range(1, N_DEV):
            peer = (my + k_) % N_DEV
            pltpu.make_async_remote_copy(
                src_ref=pbuf_ref.at[0],
                dst_ref=rs_ref.at[peer],
                send_sem=rs_send.at[0],
                recv_sem=rs_recv.at[peer],
                device_id=(my,),
                device_id_type=pl.DeviceIdType.MESH,
            ).wait_recv()

        reduced = (
            (pbuf_ref[my].astype(f32) + rs_ref[(my + 1) % N_DEV].astype(f32))
            + (rs_ref[(my + 2) % N_DEV].astype(f32)
               + rs_ref[(my + 3) % N_DEV].astype(f32))
        )
        ag_ref[my] = reduced.astype(bf16)

        ag_rdmas = []
        for k_ in range(1, N_DEV):
            peer = (my + k_) % N_DEV
            rdma = pltpu.make_async_remote_copy(
                src_ref=ag_ref.at[my],
                dst_ref=ag_ref.at[my],
                send_sem=ag_send.at[k_ - 1],
                recv_sem=ag_recv.at[my],
                device_id=(peer,),
                device_id_type=pl.DeviceIdType.MESH,
            )
            rdma.start()
            ag_rdmas.append(rdma)

        for k_ in range(1, N_DEV):
            peer = (my + k_) % N_DEV
            pltpu.make_async_remote_copy(
                src_ref=ag_ref.at[peer],
                dst_ref=ag_ref.at[peer],
                send_sem=ag_send.at[k_ - 1],
                recv_sem=ag_recv.at[peer],
                device_id=(my,),
                device_id_type=pl.DeviceIdType.MESH,
            ).wait_recv()

        for c in range(N_DEV):
            out_ref[c // CPB, (c % CPB) * CH:(c % CPB + 1) * CH, :] = (
                ag_ref[c][:, :])

        for c in range(N_DEV):
            @pl.when(my != c)
            def _():
                pltpu.make_async_remote_copy(
                    src_ref=pbuf_ref.at[c],
                    dst_ref=rs_ref.at[0],
                    send_sem=rs_send.at[c],
                    recv_sem=rs_recv.at[0],
                    device_id=(c,),
                    device_id_type=pl.DeviceIdType.MESH,
                ).wait_send()
        for rdma in ag_rdmas:
            rdma.wait_send()

    out = pl.pallas_call(
        body,
        out_shape=jax.ShapeDtypeStruct((B, SQ, D), jnp.bfloat16),
        in_specs=[pl.BlockSpec(memory_space=pltpu.VMEM)] * 5,
        out_specs=pl.BlockSpec(memory_space=pltpu.VMEM),
        scratch_shapes=[
            pltpu.VMEM((n_rows, d_local), jnp.bfloat16),
            pltpu.VMEM((4 * SQ, 4 * SQ), jnp.bfloat16),
            pltpu.VMEM((N_DEV, CH, D), jnp.bfloat16),
            pltpu.VMEM((N_DEV, CH, D), jnp.bfloat16),
            pltpu.VMEM((N_DEV, CH, D), jnp.bfloat16),
            pltpu.SemaphoreType.DMA((N_DEV,)),
            pltpu.SemaphoreType.DMA((N_DEV,)),
            pltpu.SemaphoreType.DMA((N_DEV - 1,)),
            pltpu.SemaphoreType.DMA((N_DEV,)),
        ],
        compiler_params=pltpu.CompilerParams(collective_id=0),
    )(x, Wq_b, Wk_b, Wv_b, Wo_b)

    return out
